# baseline (speedup 1.0000x reference)
import functools

import jax
import jax.numpy as jnp
from jax import lax
from jax.experimental import pallas as pl
from jax.experimental.pallas import tpu as pltpu
from jax.experimental.pallas import tpu_sc as plsc

N_ATOMS = 1000
NA = 1024
NC, NS = 2, 16
NW = NC * NS
APW = NA // NW
L = 16
M = 32
CH = 32
NB = 16
CUTOFF = 5.0

_HALF_PI_OVER_CUT = 3.14159265358979 / (2.0 * CUTOFF)


def _rsqrt16(x):
    i = lax.bitcast_convert_type(x, jnp.int32)
    i = jnp.int32(0x5F3759DF) - lax.shift_right_arithmetic(i, 1)
    y = lax.bitcast_convert_type(i, jnp.float32)
    for _ in range(2):
        y = y * (1.5 - 0.5 * x * y * y)
    return y


def _cos16(u):
    u2 = u * u
    return 1.0 + u2 * (-0.5 + u2 * (1.0 / 24.0 + u2 * (-1.0 / 720.0
           + u2 * (1.0 / 40320.0 - u2 * (1.0 / 3628800.0)))))


def _sc_body(coord_h, nbr_h, wmu_h, feat_h,
             out0_h, out1_h, out2_h,
             coord_v, nbr_v, wmu_v, feat_v,
             out0_v, out1_v, out2_v):
    wid = lax.axis_index("s") * NC + lax.axis_index("c")
    pltpu.sync_copy(coord_h, coord_v)
    pltpu.sync_copy(nbr_h.at[wid], nbr_v)
    pltpu.sync_copy(wmu_h, wmu_v)
    pltpu.sync_copy(feat_h.at[wid], feat_v)

    for g in range(APW // L):
        lb = g * L
        gbase = wid * APW + lb
        cx = coord_v[pl.ds(gbase, L)]
        cy = coord_v[pl.ds(NA + gbase, L)]
        cz = coord_v[pl.ds(2 * NA + gbase, L)]

        def m_body(m, acc):
            f0, f1x, f1y, f1z, fxx, fxy, fxz, fyy, fyz, fzz = acc
            idx = nbr_v[m, pl.ds(lb, L)]
            gx = plsc.load_gather(coord_v, [idx])
            gy = plsc.load_gather(coord_v, [idx + NA])
            gz = plsc.load_gather(coord_v, [idx + 2 * NA])
            rx = gx - cx
            ry = gy - cy
            rz = gz - cz
            d2 = rx * rx + ry * ry + rz * rz + 1e-10
            rinv = _rsqrt16(d2)
            d = d2 * rinv
            cu = _cos16(jnp.minimum(d, CUTOFF) * _HALF_PI_OVER_CUT)
            fc = cu * cu
            bsum = jnp.zeros((L,), jnp.float32)
            for b in range(NB):
                t = d - wmu_v[0, b, :]
                bsum = bsum + wmu_v[1, b, :] * jnp.exp(-(t * t))
            fn = bsum * fc
            fnx = fn * rx
            fny = fn * ry
            fnz = fn * rz
            return (f0 + fn, f1x + fnx, f1y + fny, f1z + fnz,
                    fxx + fnx * rx, fxy + fnx * ry, fxz + fnx * rz,
                    fyy + fny * ry, fyz + fny * rz, fzz + fnz * rz)

        z = jnp.zeros((L,), jnp.float32)
        F0, F1x, F1y, F1z, Fxx, Fxy, Fxz, Fyy, Fyz, Fzz = lax.fori_loop(
            0, M, m_body, (z, z, z, z, z, z, z, z, z, z))
        F1 = (F1x, F1y, F1z)
        F2 = ((Fxx, Fxy, Fxz), (Fxy, Fyy, Fyz), (Fxz, Fyz, Fzz))

        def ch_body(ch, _):
            a0 = feat_v[ch, pl.ds(lb, L)]
            a1 = [feat_v[CH + ch * 3 + p, pl.ds(lb, L)] for p in range(3)]
            a2 = [[feat_v[4 * CH + ch * 9 + 3 * p + q, pl.ds(lb, L)]
                   for q in range(3)] for p in range(3)]
            o0 = a0 * F0
            for p in range(3):
                o0 = o0 + a1[p] * F1[p]
                for q in range(3):
                    o0 = o0 + a2[p][q] * F2[p][q]
            out0_v[ch, pl.ds(lb, L)] = o0
            for p in range(3):
                o1 = a0 * F1[p] + a1[p] * F0
                for k in range(3):
                    o1 = o1 + a1[k] * F2[k][p] + a2[p][k] * F1[k]
                out1_v[p, ch, pl.ds(lb, L)] = o1
            for p in range(3):
                for q in range(3):
                    o2 = a0 * F2[p][q] + a1[p] * F1[q] + a2[p][q] * F0
                    for k in range(3):
                        o2 = o2 + a2[p][k] * F2[k][q]
                    out2_v[3 * p + q, ch, pl.ds(lb, L)] = o2
            return 0

        lax.fori_loop(0, CH, ch_body, 0)

    pltpu.sync_copy(out0_v, out0_h.at[wid])
    pltpu.sync_copy(out1_v, out1_h.at[wid])
    pltpu.sync_copy(out2_v, out2_h.at[wid])


@functools.partial(
    pl.kernel,
    out_type=(
        jax.ShapeDtypeStruct((NW, CH, APW), jnp.float32),
        jax.ShapeDtypeStruct((NW, 3, CH, APW), jnp.float32),
        jax.ShapeDtypeStruct((NW, 9, CH, APW), jnp.float32),
    ),
    mesh=plsc.VectorSubcoreMesh(core_axis_name="c", subcore_axis_name="s"),
    compiler_params=pltpu.CompilerParams(needs_layout_passes=False),
    scratch_types=[
        pltpu.VMEM((3 * NA,), jnp.float32),
        pltpu.VMEM((M, APW), jnp.int32),
        pltpu.VMEM((2, NB, L), jnp.float32),
        pltpu.VMEM((13 * CH, APW), jnp.float32),
        pltpu.VMEM((CH, APW), jnp.float32),
        pltpu.VMEM((3, CH, APW), jnp.float32),
        pltpu.VMEM((9, CH, APW), jnp.float32),
    ],
)
def _sc_kernel(coord_h, nbr_h, wmu_h, feat_h,
               out0_h, out1_h, out2_h,
               coord_v, nbr_v, wmu_v, feat_v,
               out0_v, out1_v, out2_v):
    _sc_body(coord_h, nbr_h, wmu_h, feat_h,
             out0_h, out1_h, out2_h,
             coord_v, nbr_v, wmu_v, feat_v,
             out0_v, out1_v, out2_v)


def kernel(input_tensors_0, input_tensors_1, input_tensors_2,
           coordinate, neighbor, mask, rbf_w, rbf_mu):
    pad = NA - N_ATOMS
    coord = jnp.pad(coordinate[0], ((0, pad), (0, 0)))
    coord_t = coord.T.reshape(3 * NA)
    nbr = jnp.pad(neighbor[0], ((0, pad), (0, 0)))
    nbr_c = nbr.reshape(NW, APW, M).transpose(0, 2, 1)
    feat = jnp.concatenate([
        input_tensors_0[0],
        input_tensors_1[0].reshape(N_ATOMS, CH * 3),
        input_tensors_2[0].reshape(N_ATOMS, CH * 9),
    ], axis=1)
    feat_c = (jnp.pad(feat, ((0, pad), (0, 0)))
              .reshape(NW, APW, 13 * CH).transpose(0, 2, 1))
    wmu = jnp.stack([
        jnp.tile(rbf_mu[:, None], (1, L)),
        jnp.tile(rbf_w[:, None], (1, L)),
    ]).astype(jnp.float32)

    out0_c, out1_c, out2_c = _sc_kernel(coord_t, nbr_c, wmu, feat_c)

    out0 = out0_c.transpose(0, 2, 1).reshape(NA, CH)[:N_ATOMS][None]
    out1 = out1_c.transpose(0, 3, 2, 1).reshape(NA, CH, 3)[:N_ATOMS][None]
    out2 = (out2_c.transpose(0, 3, 2, 1).reshape(NA, CH, 9)[:N_ATOMS]
            .reshape(N_ATOMS, CH, 3, 3)[None])
    return (out0, out1, out2)


# speedup vs baseline: 1.0410x; 1.0410x over previous
import functools

import jax
import jax.numpy as jnp
from jax import lax
from jax.experimental import pallas as pl
from jax.experimental.pallas import tpu as pltpu
from jax.experimental.pallas import tpu_sc as plsc

N_ATOMS = 1000
NA = 1024
NC, NS = 2, 16
NW = NC * NS
APW = NA // NW
L = 16
M = 32
CH = 32
NB = 16
CUTOFF = 5.0

_HALF_PI_OVER_CUT = 3.14159265358979 / (2.0 * CUTOFF)


def _rsqrt16(x):
    i = lax.bitcast_convert_type(x, jnp.int32)
    i = jnp.int32(0x5F3759DF) - lax.shift_right_arithmetic(i, 1)
    y = lax.bitcast_convert_type(i, jnp.float32)
    for _ in range(2):
        y = y * (1.5 - 0.5 * x * y * y)
    return y


def _cos16(u):
    u2 = u * u
    return 1.0 + u2 * (-0.5 + u2 * (1.0 / 24.0 + u2 * (-1.0 / 720.0
           + u2 * (1.0 / 40320.0 - u2 * (1.0 / 3628800.0)))))


def _sc_body(coord_h, nbr_h, wmu_h, feat_h, outf_h,
             coord_v, nbr_v, wmu_v, feat_v, outf_v):
    wid = lax.axis_index("s") * NC + lax.axis_index("c")
    pltpu.sync_copy(coord_h, coord_v)
    pltpu.sync_copy(nbr_h.at[wid], nbr_v)
    pltpu.sync_copy(wmu_h, wmu_v)
    pltpu.sync_copy(feat_h.at[wid], feat_v)

    for g in range(APW // L):
        lb = g * L
        gbase = wid * APW + lb
        cx = coord_v[pl.ds(gbase, L)]
        cy = coord_v[pl.ds(NA + gbase, L)]
        cz = coord_v[pl.ds(2 * NA + gbase, L)]

        def m_body(m, acc):
            f0, f1x, f1y, f1z, fxx, fxy, fxz, fyy, fyz, fzz = acc
            idx = nbr_v[m, pl.ds(lb, L)]
            gx = plsc.load_gather(coord_v, [idx])
            gy = plsc.load_gather(coord_v, [idx + NA])
            gz = plsc.load_gather(coord_v, [idx + 2 * NA])
            rx = gx - cx
            ry = gy - cy
            rz = gz - cz
            d2 = rx * rx + ry * ry + rz * rz + 1e-10
            rinv = _rsqrt16(d2)
            d = d2 * rinv
            cu = _cos16(jnp.minimum(d, CUTOFF) * _HALF_PI_OVER_CUT)
            fc = cu * cu
            bsum = jnp.zeros((L,), jnp.float32)
            for b in range(NB):
                t = d - wmu_v[0, b, :]
                bsum = bsum + wmu_v[1, b, :] * jnp.exp(-(t * t))
            fn = bsum * fc
            fnx = fn * rx
            fny = fn * ry
            fnz = fn * rz
            return (f0 + fn, f1x + fnx, f1y + fny, f1z + fnz,
                    fxx + fnx * rx, fxy + fnx * ry, fxz + fnx * rz,
                    fyy + fny * ry, fyz + fny * rz, fzz + fnz * rz)

        z = jnp.zeros((L,), jnp.float32)
        F0, F1x, F1y, F1z, Fxx, Fxy, Fxz, Fyy, Fyz, Fzz = lax.fori_loop(
            0, M, m_body, (z, z, z, z, z, z, z, z, z, z))
        F1 = (F1x, F1y, F1z)
        F2 = ((Fxx, Fxy, Fxz), (Fxy, Fyy, Fyz), (Fxz, Fyz, Fzz))

        def ch_body(ch, _):
            a0 = feat_v[ch, pl.ds(lb, L)]
            a1 = [feat_v[CH + ch * 3 + p, pl.ds(lb, L)] for p in range(3)]
            a2 = [[feat_v[4 * CH + ch * 9 + 3 * p + q, pl.ds(lb, L)]
                   for q in range(3)] for p in range(3)]
            o0 = a0 * F0
            for p in range(3):
                o0 = o0 + a1[p] * F1[p]
                for q in range(3):
                    o0 = o0 + a2[p][q] * F2[p][q]
            outf_v[ch, pl.ds(lb, L)] = o0
            for p in range(3):
                o1 = a0 * F1[p] + a1[p] * F0
                for k in range(3):
                    o1 = o1 + a1[k] * F2[k][p] + a2[p][k] * F1[k]
                outf_v[CH + ch * 3 + p, pl.ds(lb, L)] = o1
            for p in range(3):
                for q in range(3):
                    o2 = a0 * F2[p][q] + a1[p] * F1[q] + a2[p][q] * F0
                    for k in range(3):
                        o2 = o2 + a2[p][k] * F2[k][q]
                    outf_v[4 * CH + ch * 9 + 3 * p + q, pl.ds(lb, L)] = o2
            return 0

        lax.fori_loop(0, CH, ch_body, 0)

    pltpu.sync_copy(outf_v, outf_h.at[wid])


@functools.partial(
    pl.kernel,
    out_type=jax.ShapeDtypeStruct((NW, 13 * CH, APW), jnp.float32),
    mesh=plsc.VectorSubcoreMesh(core_axis_name="c", subcore_axis_name="s"),
    compiler_params=pltpu.CompilerParams(needs_layout_passes=False),
    scratch_types=[
        pltpu.VMEM((3 * NA,), jnp.float32),
        pltpu.VMEM((M, APW), jnp.int32),
        pltpu.VMEM((2, NB, L), jnp.float32),
        pltpu.VMEM((13 * CH, APW), jnp.float32),
        pltpu.VMEM((13 * CH, APW), jnp.float32),
    ],
)
def _sc_kernel(coord_h, nbr_h, wmu_h, feat_h, outf_h,
               coord_v, nbr_v, wmu_v, feat_v, outf_v):
    _sc_body(coord_h, nbr_h, wmu_h, feat_h, outf_h,
             coord_v, nbr_v, wmu_v, feat_v, outf_v)


def kernel(input_tensors_0, input_tensors_1, input_tensors_2,
           coordinate, neighbor, mask, rbf_w, rbf_mu):
    pad = NA - N_ATOMS
    coord = jnp.pad(coordinate[0], ((0, pad), (0, 0)))
    coord_t = coord.T.reshape(3 * NA)
    nbr = jnp.pad(neighbor[0], ((0, pad), (0, 0)))
    nbr_c = nbr.reshape(NW, APW, M).transpose(0, 2, 1)
    feat = jnp.concatenate([
        input_tensors_0[0],
        input_tensors_1[0].reshape(N_ATOMS, CH * 3),
        input_tensors_2[0].reshape(N_ATOMS, CH * 9),
    ], axis=1)
    feat_c = (jnp.pad(feat, ((0, pad), (0, 0)))
              .reshape(NW, APW, 13 * CH).transpose(0, 2, 1))
    wmu = jnp.stack([
        jnp.tile(rbf_mu[:, None], (1, L)),
        jnp.tile(rbf_w[:, None], (1, L)),
    ]).astype(jnp.float32)

    outf_c = _sc_kernel(coord_t, nbr_c, wmu, feat_c)

    outf = outf_c.transpose(0, 2, 1).reshape(NA, 13 * CH)[:N_ATOMS]
    out0 = outf[:, :CH][None]
    out1 = outf[:, CH:4 * CH].reshape(N_ATOMS, CH, 3)[None]
    out2 = outf[:, 4 * CH:].reshape(N_ATOMS, CH, 3, 3)[None]
    return (out0, out1, out2)
